# baseline (speedup 1.0000x reference)
import functools

import jax
import jax.numpy as jnp
from jax import lax
from jax.experimental import pallas as pl
from jax.experimental.pallas import tpu as pltpu
from jax.experimental.pallas import tpu_sc as plsc

_NC = 2
_NS = 16
_NW = _NC * _NS
_C = 128
_NBUF = 6


@functools.partial(jax.jit, static_argnames=("cpw", "dim"))
def _gather_sc(idx3, table, cpw, dim):
    n = _NW * cpw * _C
    m = cpw
    assert m % 2 == 0 and m > _NBUF
    mesh = plsc.VectorSubcoreMesh(core_axis_name="c", subcore_axis_name="s")

    @functools.partial(
        pl.kernel,
        out_type=jax.ShapeDtypeStruct((n // _C, _C, dim), table.dtype),
        mesh=mesh,
        scratch_types=[
            pltpu.VMEM((cpw, _C), jnp.int32),
            pltpu.VMEM((_NBUF, _C, dim), table.dtype),
            pltpu.SemaphoreType.DMA((_NBUF,)),
            pltpu.SemaphoreType.DMA((_NBUF // 2,)),
        ],
    )
    def k(idx_hbm, table_hbm, out_hbm, idx_v, rows_v, gsem, wsem):
        wid = lax.axis_index("s") * _NC + lax.axis_index("c")
        pltpu.sync_copy(idx_hbm.at[wid], idx_v)
        base = wid * cpw

        def fire_gather(j, b):
            pltpu.async_copy(
                table_hbm.at[idx_v.at[j]], rows_v.at[b], gsem.at[b])

        def wait_gather(j, b):
            pltpu.make_async_copy(
                table_hbm.at[idx_v.at[j]], rows_v.at[b], gsem.at[b]).wait()

        def fire_write(j0, s0):
            pltpu.async_copy(
                rows_v.at[pl.ds(s0, 2)], out_hbm.at[pl.ds(base + j0, 2)],
                wsem.at[s0 // 2])

        def wait_write(j0, s0):
            pltpu.make_async_copy(
                rows_v.at[pl.ds(s0, 2)], out_hbm.at[pl.ds(base + j0, 2)],
                wsem.at[s0 // 2]).wait()

        for b in range(_NBUF):
            fire_gather(b, b)
        prev = None
        for jp in range(m // 2):
            j0 = 2 * jp
            s0 = j0 % _NBUF
            wait_gather(j0, s0)
            wait_gather(j0 + 1, s0 + 1)
            fire_write(j0, s0)
            if prev is not None:
                pj, ps = prev
                wait_write(pj, ps)
                if pj + _NBUF < m:
                    fire_gather(pj + _NBUF, ps)
                if pj + _NBUF + 1 < m:
                    fire_gather(pj + _NBUF + 1, ps + 1)
            prev = (j0, s0)
        wait_write(*prev)

    return k(idx3, table)


def kernel(indices, table):
    b, l = indices.shape
    dim = table.shape[1]
    n = b * l
    assert n % (_NW * _C) == 0
    cpw = n // (_NW * _C)
    idx3 = indices.T.reshape(_NW, cpw, _C)
    out = _gather_sc(idx3, table, cpw, dim)
    return out.reshape(l, b, dim).transpose(1, 0, 2)


# speedup vs baseline: 1.0576x; 1.0576x over previous
import functools

import jax
import jax.numpy as jnp
from jax import lax
from jax.experimental import pallas as pl
from jax.experimental.pallas import tpu as pltpu
from jax.experimental.pallas import tpu_sc as plsc

_NC = 2
_NS = 16
_NW = _NC * _NS
_C = 128
_NBUF = 5


@functools.partial(jax.jit, static_argnames=("bsz", "lsz", "dim"))
def _gather_sc(idx_t, table, bsz, lsz, dim):
    n = bsz * lsz
    m = lsz
    assert bsz == _NW * _C and m % _NBUF == 0 and m >= 2 * _NBUF
    mesh = plsc.VectorSubcoreMesh(core_axis_name="c", subcore_axis_name="s")

    @functools.partial(
        pl.kernel,
        out_type=jax.ShapeDtypeStruct((n, dim), table.dtype),
        mesh=mesh,
        scratch_types=[
            pltpu.VMEM((m, _C), jnp.int32),
            pltpu.VMEM((_NBUF, _C, dim), table.dtype),
            pltpu.SemaphoreType.DMA((_NBUF,)),
            pltpu.SemaphoreType.DMA((_NBUF,)),
        ],
    )
    def k(idx_hbm, table_hbm, out_hbm, idx_v, rows_v, gsem, wsem):
        wid = lax.axis_index("s") * _NC + lax.axis_index("c")
        col = wid * _C
        pltpu.sync_copy(idx_hbm.at[:, pl.ds(col, _C)], idx_v)

        def fire_gather(j, b):
            pltpu.async_copy(
                table_hbm.at[idx_v.at[j]], rows_v.at[b], gsem.at[b])

        def wait_gather(j, b):
            pltpu.make_async_copy(
                table_hbm.at[idx_v.at[j]], rows_v.at[b], gsem.at[b]).wait()

        def fire_write(j, b):
            pltpu.async_copy(
                rows_v.at[b], out_hbm.at[pl.ds(j * bsz + col, _C)],
                wsem.at[b])

        def wait_write(j, b):
            pltpu.make_async_copy(
                rows_v.at[b], out_hbm.at[pl.ds(j * bsz + col, _C)],
                wsem.at[b]).wait()

        for b in range(_NBUF):
            fire_gather(b, b)

        @pl.loop(0, m - _NBUF, step=_NBUF)
        def _(j0):
            for b in range(_NBUF):
                j = j0 + b
                wait_gather(j, b)
                fire_write(j, b)
                wait_write(j, b)
                fire_gather(j + _NBUF, b)

        for b in range(_NBUF):
            j = m - _NBUF + b
            wait_gather(j, b)
            fire_write(j, b)
        for b in range(_NBUF):
            wait_write(m - _NBUF + b, b)

    return k(idx_t, table)


def kernel(indices, table):
    b, l = indices.shape
    dim = table.shape[1]
    out = _gather_sc(indices.T, table, b, l, dim)
    return out.reshape(l, b, dim).transpose(1, 0, 2)


# speedup vs baseline: 1.0596x; 1.0019x over previous
import functools

import jax
import jax.numpy as jnp
from jax import lax
from jax.experimental import pallas as pl
from jax.experimental.pallas import tpu as pltpu
from jax.experimental.pallas import tpu_sc as plsc

_NC = 2
_NS = 16
_NW = _NC * _NS
_C = 128
_NBUF = 5


@functools.partial(jax.jit, static_argnames=("bsz", "lsz", "dim"))
def _gather_sc(idx_t, table, bsz, lsz, dim):
    n = bsz * lsz
    m = lsz
    assert bsz == _NW * _C and m % _NBUF == 0 and m >= 2 * _NBUF
    mesh = plsc.VectorSubcoreMesh(core_axis_name="c", subcore_axis_name="s")

    @functools.partial(
        pl.kernel,
        out_type=jax.ShapeDtypeStruct((n, dim), table.dtype),
        mesh=mesh,
        compiler_params=pltpu.CompilerParams(
            disable_bounds_checks=True,
            disable_semaphore_checks=True,
        ),
        scratch_types=[
            pltpu.VMEM((m, _C), jnp.int32),
            pltpu.VMEM((_NBUF, _C, dim), table.dtype),
            pltpu.SemaphoreType.DMA((_NBUF,)),
            pltpu.SemaphoreType.DMA((_NBUF,)),
        ],
    )
    def k(idx_hbm, table_hbm, out_hbm, idx_v, rows_v, gsem, wsem):
        wid = lax.axis_index("s") * _NC + lax.axis_index("c")
        col = wid * _C
        pltpu.sync_copy(idx_hbm.at[:, pl.ds(col, _C)], idx_v)

        def fire_gather(j, b):
            pltpu.async_copy(
                table_hbm.at[idx_v.at[j]], rows_v.at[b], gsem.at[b])

        def wait_gather(j, b):
            pltpu.make_async_copy(
                table_hbm.at[idx_v.at[j]], rows_v.at[b], gsem.at[b]).wait()

        def fire_write(j, b):
            pltpu.async_copy(
                rows_v.at[b], out_hbm.at[pl.ds(j * bsz + col, _C)],
                wsem.at[b])

        def wait_write(j, b):
            pltpu.make_async_copy(
                rows_v.at[b], out_hbm.at[pl.ds(j * bsz + col, _C)],
                wsem.at[b]).wait()

        for b in range(_NBUF):
            fire_gather(b, b)

        @pl.loop(0, m - _NBUF, step=_NBUF)
        def _(j0):
            for b in range(_NBUF):
                j = j0 + b
                wait_gather(j, b)
                fire_write(j, b)
                wait_write(j, b)
                fire_gather(j + _NBUF, b)

        for b in range(_NBUF):
            j = m - _NBUF + b
            wait_gather(j, b)
            fire_write(j, b)
        for b in range(_NBUF):
            wait_write(m - _NBUF + b, b)

    return k(idx_t, table)


def kernel(indices, table):
    b, l = indices.shape
    dim = table.shape[1]
    out = _gather_sc(indices.T, table, b, l, dim)
    return out.reshape(l, b, dim).transpose(1, 0, 2)
